# baseline (speedup 1.0000x reference)
import functools

import jax
import jax.numpy as jnp
from jax import lax
from jax.experimental import pallas as pl
from jax.experimental.pallas import tpu as pltpu
from jax.experimental.pallas import tpu_sc as plsc

N = 100000
NP = 100352
IL = NP // 8
ILB = IL // 8
E = 1600000
G = 512
L = 16
NSUB = 16
NCORE = 2
EB = 400
NPT = N // NSUB
NB0 = 122
NB1 = 128
NBUF = 4

_MESH = dict(core_axis_name="c", subcore_axis_name="s")

_SELU_SCALE = 1.0507009873554805
_SELU_ALPHA = 1.6732632423543772


def _selu(t):
    neg = _SELU_ALPHA * (jnp.exp(jnp.minimum(t, 0.0)) - 1.0)
    return _SELU_SCALE * jnp.where(t > 0.0, t, neg)


def _bd(M):
    return jnp.kron(jnp.eye(8, dtype=jnp.float32), M)


def _bd_expand(W, cin, cout):
    return jnp.stack([
        jnp.stack([_bd(W[16 * c:16 * c + 16, 16 * p:16 * p + 16])
                   for p in range(cout)])
        for c in range(cin)])


def _tile8(b):
    c = b.shape[0] // 16
    return jnp.tile(b.reshape(c, 1, 16), (1, 8, 1)).reshape(c, 128)


def _edge_layout(cid, sid):
    nb = NB0 + (NB1 - NB0) * cid
    base = cid * (NSUB * NB0 * EB) + sid * (nb * EB)
    return nb, base


def _zero_slice(rows, zsh, row0):
    for j in range(NPT // EB):
        pltpu.sync_copy(rows, zsh.at[pl.ds(row0 + j * EB, EB), :])
    rem = NPT % EB
    pltpu.sync_copy(rows.at[pl.ds(0, rem), :],
                    zsh.at[pl.ds(row0 + (NPT // EB) * EB, rem), :])


def _sc_deg(dst):
    ones = jnp.ones((EB, L), jnp.float32)
    zeros = jnp.zeros((EB, L), jnp.float32)

    @functools.partial(
        pl.kernel,
        out_type=jax.ShapeDtypeStruct((NCORE, NP, L), jnp.float32),
        mesh=plsc.VectorSubcoreMesh(**_MESH),
        compiler_params=pltpu.CompilerParams(use_tc_tiling_on_sc=False),
        scratch_types=[
            pltpu.VMEM((EB,), jnp.int32),
            pltpu.VMEM((EB,), jnp.int32),
            pltpu.VMEM((EB, L), jnp.float32),
            pltpu.VMEM((EB, L), jnp.float32),
            pltpu.VMEM_SHARED((NP, L), jnp.float32),
            pltpu.SemaphoreType.DMA,
            pltpu.SemaphoreType.DMA,
        ],
    )
    def run(dst_r, ones_r, zeros_r, out_r, didx0, didx1, ones_v, zbuf, zsh,
            sem0, sem1):
        cid = lax.axis_index("c")
        sid = lax.axis_index("s")
        row0 = sid * NPT
        nb, base = _edge_layout(cid, sid)
        pltpu.sync_copy(ones_r, ones_v)
        pltpu.sync_copy(zeros_r, zbuf)
        _zero_slice(zbuf, zsh, row0)

        @pl.when(sid == NSUB - 1)
        def _():
            pltpu.sync_copy(zbuf.at[pl.ds(0, NP - N), :],
                            zsh.at[pl.ds(N, NP - N), :])

        plsc.subcore_barrier()

        didx = (didx0, didx1)
        sems = (sem0, sem1)

        pltpu.sync_copy(dst_r.at[pl.ds(base, EB)], didx0)
        pltpu.async_copy(ones_v, zsh.at[didx0], sem0, add=True)

        def ring(g, carry):
            for j in range(2):
                b = 2 * g + j
                q = (j + 1) % 2

                @pl.when(b + 1 < nb)
                def _():
                    off = pl.multiple_of(base + (b + 1) * EB, EB)
                    pltpu.sync_copy(dst_r.at[pl.ds(off, EB)], didx[q])
                    pltpu.async_copy(ones_v, zsh.at[didx[q]], sems[q],
                                     add=True)
                pltpu.make_async_copy(ones_v, zsh.at[didx[j]], sems[j]).wait()
            return carry

        lax.fori_loop(0, nb // 2, ring, 0)

        @pl.when(nb % 2 == 1)
        def _():
            pltpu.make_async_copy(ones_v, zsh.at[didx0], sem0).wait()

        plsc.subcore_barrier()
        pltpu.sync_copy(zsh.at[pl.ds(row0, NPT), :],
                        out_r.at[cid, pl.ds(row0, NPT), :])

        @pl.when(sid == NSUB - 1)
        def _():
            pltpu.sync_copy(zbuf.at[pl.ds(0, NP - N), :],
                            out_r.at[cid, pl.ds(N, NP - N), :])

    return run(dst, ones, zeros)


def _sc_edge_pass(tables, src, dst):
    C = len(tables)
    zeros = jnp.zeros((EB, L), jnp.float32)

    @functools.partial(
        pl.kernel,
        out_type=jax.ShapeDtypeStruct((NCORE, C, NP, L), jnp.float32),
        mesh=plsc.VectorSubcoreMesh(**_MESH),
        compiler_params=pltpu.CompilerParams(use_tc_tiling_on_sc=False),
        scratch_types=(
            [pltpu.VMEM((EB,), jnp.int32)] * (2 * NBUF)
            + [pltpu.VMEM((EB, L), jnp.float32)] * NBUF
            + [pltpu.VMEM_SHARED((NP, L), jnp.float32)]
            + [pltpu.SemaphoreType.DMA] * (2 * NBUF)
        ),
    )
    def run(*refs):
        t_refs = refs[:C]
        src_r, dst_r, zeros_r, out_r = refs[C], refs[C + 1], refs[C + 2], refs[C + 3]
        sidx = refs[C + 4:C + 4 + NBUF]
        didx = refs[C + 4 + NBUF:C + 4 + 2 * NBUF]
        rows = refs[C + 4 + 2 * NBUF:C + 4 + 3 * NBUF]
        zsh = refs[C + 4 + 3 * NBUF]
        sg = refs[C + 5 + 3 * NBUF:C + 5 + 3 * NBUF + NBUF]
        ss = refs[C + 5 + 4 * NBUF:C + 5 + 5 * NBUF]
        cid = lax.axis_index("c")
        sid = lax.axis_index("s")
        row0 = sid * NPT
        nb, base = _edge_layout(cid, sid)

        for c in range(C):
            tab = t_refs[c]
            pltpu.sync_copy(zeros_r, rows[0])
            _zero_slice(rows[0], zsh, row0)
            plsc.subcore_barrier()

            def idx_copy(b, j):
                off = pl.multiple_of(base + b * EB, EB)
                pltpu.sync_copy(src_r.at[pl.ds(off, EB)], sidx[j])
                pltpu.sync_copy(dst_r.at[pl.ds(off, EB)], didx[j])

            for j in range(NBUF):
                idx_copy(j, j)
                pltpu.async_copy(tab.at[sidx[j]], rows[j], sg[j])

            def ring(g, carry, tab=tab, idx_copy=idx_copy):
                for j in range(NBUF):
                    b = NBUF * g + j

                    @pl.when(b < nb)
                    def _(tab=tab, idx_copy=idx_copy, j=j, b=b):
                        pltpu.make_async_copy(tab.at[sidx[j]], rows[j],
                                              sg[j]).wait()
                        pltpu.async_copy(rows[j], zsh.at[didx[j]], ss[j],
                                         add=True)
                        pltpu.make_async_copy(rows[j], zsh.at[didx[j]],
                                              ss[j]).wait()

                        @pl.when(b + NBUF < nb)
                        def _(tab=tab, idx_copy=idx_copy, j=j, b=b):
                            idx_copy(b + NBUF, j)
                            pltpu.async_copy(tab.at[sidx[j]], rows[j], sg[j])
                return carry

            lax.fori_loop(0, (nb + NBUF - 1) // NBUF, ring, 0)
            plsc.subcore_barrier()
            pltpu.sync_copy(zsh.at[pl.ds(row0, NPT), :],
                            out_r.at[cid, c, pl.ds(row0, NPT), :])

    return run(*tables, src, dst, zeros)


def _sc_pool(h_chunks, batch_vec):
    C = len(h_chunks)
    NBP = N // EB
    PER = -(-NBP // (NCORE * NSUB))
    GPT = G // NSUB
    zeros = jnp.zeros((EB, L), jnp.float32)

    @functools.partial(
        pl.kernel,
        out_type=jax.ShapeDtypeStruct((NCORE, C, G, L), jnp.float32),
        mesh=plsc.VectorSubcoreMesh(**_MESH),
        compiler_params=pltpu.CompilerParams(use_tc_tiling_on_sc=False),
        scratch_types=[
            pltpu.VMEM((EB,), jnp.int32),
            pltpu.VMEM((EB, L), jnp.float32),
            [pltpu.VMEM_SHARED((G, L), jnp.float32) for _ in range(C)],
        ],
    )
    def run(*refs):
        h_refs = refs[:C]
        bv_r, zeros_r, out_r = refs[C], refs[C + 1], refs[C + 2]
        didx, rows = refs[C + 3], refs[C + 4]
        zshs = refs[C + 5]
        cid = lax.axis_index("c")
        sid = lax.axis_index("s")
        wid = cid * NSUB + sid
        grow0 = sid * GPT
        pltpu.sync_copy(zeros_r, rows)
        for c in range(C):
            pltpu.sync_copy(rows.at[pl.ds(0, GPT), :],
                            zshs[c].at[pl.ds(grow0, GPT), :])
        plsc.subcore_barrier()
        for t in range(PER):
            b = wid + t * NCORE * NSUB

            @pl.when(b < NBP)
            def _():
                base = pl.multiple_of(b * EB, EB)
                pltpu.sync_copy(bv_r.at[pl.ds(base, EB)], didx)
                for c in range(C):
                    pltpu.sync_copy(h_refs[c].at[pl.ds(base, EB), :], rows)
                    pltpu.sync_copy(rows, zshs[c].at[didx], add=True)

        plsc.subcore_barrier()
        for c in range(C):
            pltpu.sync_copy(zshs[c].at[pl.ds(grow0, GPT), :],
                            out_r.at[cid, c, pl.ds(grow0, GPT), :])

    return run(*h_chunks, batch_vec, zeros)


def _tc_prep(x_p3, W1p, deg_ilv):
    def body(x_ref, w_ref, deg_ref, y_ref, dis_ref):
        d = deg_ref[0] + deg_ref[1] + 1.0
        dis = lax.rsqrt(d)
        dis_ref[...] = dis
        w = w_ref[...]
        parts = [jnp.dot(x_ref[:, j, :], w, preferred_element_type=jnp.float32)
                 for j in range(8)]
        y_ref[...] = dis * jnp.concatenate(parts, axis=1)

    return pl.pallas_call(
        body,
        grid=(IL // ILB,),
        in_specs=[
            pl.BlockSpec((ILB, 8, 128), lambda i: (i, 0, 0)),
            pl.BlockSpec((128, L), lambda i: (0, 0)),
            pl.BlockSpec((NCORE, ILB, 128), lambda i: (0, i, 0)),
        ],
        out_specs=[
            pl.BlockSpec((ILB, 128), lambda i: (i, 0)),
            pl.BlockSpec((ILB, 128), lambda i: (i, 0)),
        ],
        out_shape=[
            jax.ShapeDtypeStruct((IL, 128), jnp.float32),
            jax.ShapeDtypeStruct((IL, 128), jnp.float32),
        ],
    )(x_p3, W1p, deg_ilv)


def _tc_update(z_ilv, ychunks, dis, b128, Wbd, cout, last=False):
    cin = len(ychunks)

    def body(*refs):
        z_ref = refs[0]
        y_refs = refs[1:1 + cin]
        dis_ref = refs[1 + cin]
        b_ref = refs[2 + cin]
        k = 3 + cin
        w_ref = None
        if not last:
            w_ref = refs[k]
            k += 1
        outs = refs[k:]
        dis = dis_ref[...]
        ts = [_selu(dis * (z_ref[0, c] + z_ref[1, c] + y_refs[c][...])
                    + b_ref[c]) for c in range(cin)]
        if last:
            for c in range(cout):
                outs[c][...] = ts[c]
        else:
            for p in range(cout):
                acc = jnp.dot(ts[0], w_ref[0, p],
                              preferred_element_type=jnp.float32)
                for c in range(1, cin):
                    acc = acc + jnp.dot(ts[c], w_ref[c, p],
                                        preferred_element_type=jnp.float32)
                outs[p][...] = dis * acc

    in_specs = [pl.BlockSpec((NCORE, cin, ILB, 128), lambda i: (0, 0, i, 0))]
    in_specs += [pl.BlockSpec((ILB, 128), lambda i: (i, 0))] * cin
    in_specs += [pl.BlockSpec((ILB, 128), lambda i: (i, 0)),
                 pl.BlockSpec((cin, 128), lambda i: (0, 0))]
    args = [z_ilv, *ychunks, dis, b128]
    if not last:
        in_specs.append(pl.BlockSpec((cin, cout, 128, 128),
                                     lambda i: (0, 0, 0, 0)))
        args.append(Wbd)
    outs = pl.pallas_call(
        body,
        grid=(IL // ILB,),
        in_specs=in_specs,
        out_specs=[pl.BlockSpec((ILB, 128), lambda i: (i, 0))] * cout,
        out_shape=[jax.ShapeDtypeStruct((IL, 128), jnp.float32)] * cout,
    )(*args)
    return list(outs)


def _tc_final(pool_ilv, W1bd, b1t, W2bd, b2t):
    CIN, COUT = W1bd.shape[0], W1bd.shape[1]

    def body(p_ref, w1_ref, b1_ref, w2_ref, b2_ref, o_ref):
        ps = [p_ref[0, c] + p_ref[1, c] for c in range(CIN)]
        o1 = []
        for p in range(COUT):
            acc = jnp.dot(ps[0], w1_ref[0, p], preferred_element_type=jnp.float32)
            for c in range(1, CIN):
                acc = acc + jnp.dot(ps[c], w1_ref[c, p],
                                    preferred_element_type=jnp.float32)
            o1.append(jnp.maximum(acc + b1_ref[p], 0.0))
        acc = jnp.dot(o1[0], w2_ref[0, 0], preferred_element_type=jnp.float32)
        for p in range(1, COUT):
            acc = acc + jnp.dot(o1[p], w2_ref[p, 0],
                                preferred_element_type=jnp.float32)
        o_ref[...] = jnp.maximum(acc + b2_ref[0], 0.0)

    return pl.pallas_call(
        body,
        out_shape=jax.ShapeDtypeStruct((G // 8, 128), jnp.float32),
    )(pool_ilv, W1bd, b1t, W2bd, b2t)


def kernel(x, edge_index, batch_vec, W1, b1, W2, b2, W3, b3, W4, b4,
           Wl1, bl1, Wl2, bl2):
    src = edge_index[0]
    dst = edge_index[1]
    x_p3 = jnp.pad(x, ((0, NP - N), (0, 0))).reshape(IL, 8, 128)

    W1p = jnp.pad(W1, ((0, 0), (0, 1)))
    b1t = _tile8(jnp.pad(b1, (0, 1)))
    W2bd = _bd_expand(jnp.pad(W2, ((0, 1), (0, 12))), 1, 2)
    b2t = _tile8(jnp.pad(b2, (0, 12)))
    W3bd = _bd_expand(jnp.pad(W3, ((0, 12), (0, 5))), 2, 2)
    b3t = _tile8(jnp.pad(b3, (0, 5)))
    W4bd = _bd_expand(jnp.pad(W4, ((0, 5), (0, 12))), 2, 3)
    b4t = _tile8(jnp.pad(b4, (0, 12)))
    Wl1bd = _bd_expand(jnp.pad(Wl1, ((0, 12), (0, 0))), 3, 6)
    bl1t = _tile8(bl1)
    Wl2bd = _bd_expand(jnp.pad(Wl2, ((0, 0), (0, 4))), 6, 1)
    bl2t = _tile8(jnp.pad(bl2, (0, 4)))

    deg = _sc_deg(dst)
    y1, dis = _tc_prep(x_p3, W1p, deg.reshape(NCORE, IL, 128))
    ys = [y1]
    for Wbd, bt in [(W2bd, b1t), (W3bd, b2t), (W4bd, b3t)]:
        z = _sc_edge_pass([y.reshape(NP, L) for y in ys], src, dst)
        ys = _tc_update(z.reshape(NCORE, len(ys), IL, 128), ys, dis, bt,
                        Wbd, cout=Wbd.shape[1])
    z = _sc_edge_pass([y.reshape(NP, L) for y in ys], src, dst)
    hs = _tc_update(z.reshape(NCORE, len(ys), IL, 128), ys, dis, b4t,
                    None, cout=3, last=True)
    pool = _sc_pool([h.reshape(NP, L) for h in hs], batch_vec)
    o = _tc_final(pool.reshape(NCORE, 3, G // 8, 128), Wl1bd, bl1t,
                  Wl2bd, bl2t)
    return o.reshape(G, L)[:, :12]


# speedup vs baseline: 1.3929x; 1.3929x over previous
import functools

import jax
import jax.numpy as jnp
from jax import lax
from jax.experimental import pallas as pl
from jax.experimental.pallas import tpu as pltpu
from jax.experimental.pallas import tpu_sc as plsc

N = 100000
NP = 100352
IL = NP // 8
ILB = IL // 8
E = 1600000
G = 512
L = 16
NSUB = 16
NCORE = 2
EB = 800
NPT = N // NSUB
NB0 = 61
NB1 = 64

_MESH = dict(core_axis_name="c", subcore_axis_name="s")

_SELU_SCALE = 1.0507009873554805
_SELU_ALPHA = 1.6732632423543772


def _selu(t):
    neg = _SELU_ALPHA * (jnp.exp(jnp.minimum(t, 0.0)) - 1.0)
    return _SELU_SCALE * jnp.where(t > 0.0, t, neg)


def _bd(M):
    return jnp.kron(jnp.eye(8, dtype=jnp.float32), M)


def _bd_expand(W, cin, cout):
    return jnp.stack([
        jnp.stack([_bd(W[16 * c:16 * c + 16, 16 * p:16 * p + 16])
                   for p in range(cout)])
        for c in range(cin)])


def _tile8(b):
    c = b.shape[0] // 16
    return jnp.tile(b.reshape(c, 1, 16), (1, 8, 1)).reshape(c, 128)


def _edge_layout(cid, sid):
    nb = NB0 + (NB1 - NB0) * cid
    base = cid * (NSUB * NB0 * EB) + sid * (nb * EB)
    return nb, base


def _zero_slice(rows, zsh, row0):
    for j in range(NPT // EB):
        pltpu.sync_copy(rows, zsh.at[pl.ds(row0 + j * EB, EB), :])
    rem = NPT % EB
    pltpu.sync_copy(rows.at[pl.ds(0, rem), :],
                    zsh.at[pl.ds(row0 + (NPT // EB) * EB, rem), :])


def _sc_deg(dst):
    ones = jnp.ones((EB, L), jnp.float32)
    zeros = jnp.zeros((EB, L), jnp.float32)

    @functools.partial(
        pl.kernel,
        out_type=jax.ShapeDtypeStruct((NCORE, NP, L), jnp.float32),
        mesh=plsc.VectorSubcoreMesh(**_MESH),
        compiler_params=pltpu.CompilerParams(use_tc_tiling_on_sc=False),
        scratch_types=[
            pltpu.VMEM((EB,), jnp.int32),
            pltpu.VMEM((EB,), jnp.int32),
            pltpu.VMEM((EB, L), jnp.float32),
            pltpu.VMEM((EB, L), jnp.float32),
            pltpu.VMEM_SHARED((NP, L), jnp.float32),
            pltpu.SemaphoreType.DMA,
            pltpu.SemaphoreType.DMA,
        ],
    )
    def run(dst_r, ones_r, zeros_r, out_r, didx0, didx1, ones_v, zbuf, zsh,
            sem0, sem1):
        cid = lax.axis_index("c")
        sid = lax.axis_index("s")
        row0 = sid * NPT
        nb, base = _edge_layout(cid, sid)
        pltpu.sync_copy(ones_r, ones_v)
        pltpu.sync_copy(zeros_r, zbuf)
        _zero_slice(zbuf, zsh, row0)

        @pl.when(sid == NSUB - 1)
        def _():
            pltpu.sync_copy(zbuf.at[pl.ds(0, NP - N), :],
                            zsh.at[pl.ds(N, NP - N), :])

        plsc.subcore_barrier()

        didx = (didx0, didx1)
        sems = (sem0, sem1)

        pltpu.sync_copy(dst_r.at[pl.ds(base, EB)], didx0)
        pltpu.async_copy(ones_v, zsh.at[didx0], sem0, add=True)

        def ring(g, carry):
            for j in range(2):
                b = 2 * g + j
                q = (j + 1) % 2

                @pl.when(b + 1 < nb)
                def _():
                    off = pl.multiple_of(base + (b + 1) * EB, EB)
                    pltpu.sync_copy(dst_r.at[pl.ds(off, EB)], didx[q])
                    pltpu.async_copy(ones_v, zsh.at[didx[q]], sems[q],
                                     add=True)
                pltpu.make_async_copy(ones_v, zsh.at[didx[j]], sems[j]).wait()
            return carry

        lax.fori_loop(0, nb // 2, ring, 0)

        @pl.when(nb % 2 == 1)
        def _():
            pltpu.make_async_copy(ones_v, zsh.at[didx0], sem0).wait()

        plsc.subcore_barrier()
        pltpu.sync_copy(zsh.at[pl.ds(row0, NPT), :],
                        out_r.at[cid, pl.ds(row0, NPT), :])

        @pl.when(sid == NSUB - 1)
        def _():
            pltpu.sync_copy(zbuf.at[pl.ds(0, NP - N), :],
                            out_r.at[cid, pl.ds(N, NP - N), :])

    return run(dst, ones, zeros)


def _sc_edge_pass(tables, src, dst):
    C = len(tables)
    zeros = jnp.zeros((EB, L), jnp.float32)

    @functools.partial(
        pl.kernel,
        out_type=jax.ShapeDtypeStruct((NCORE, C, NP, L), jnp.float32),
        mesh=plsc.VectorSubcoreMesh(**_MESH),
        compiler_params=pltpu.CompilerParams(use_tc_tiling_on_sc=False),
        scratch_types=[
            pltpu.VMEM((EB,), jnp.int32),
            pltpu.VMEM((EB,), jnp.int32),
            pltpu.VMEM((EB,), jnp.int32),
            pltpu.VMEM((EB,), jnp.int32),
            pltpu.VMEM((EB, L), jnp.float32),
            pltpu.VMEM((EB, L), jnp.float32),
            pltpu.VMEM_SHARED((NP, L), jnp.float32),
            pltpu.SemaphoreType.DMA,
            pltpu.SemaphoreType.DMA,
            pltpu.SemaphoreType.DMA,
            pltpu.SemaphoreType.DMA,
        ],
    )
    def run(*refs):
        t_refs = refs[:C]
        src_r, dst_r, zeros_r, out_r = refs[C], refs[C + 1], refs[C + 2], refs[C + 3]
        (sidx0, sidx1, didx0, didx1, rows0, rows1, zsh,
         sg0, sg1, ss0, ss1) = refs[C + 4:]
        sidx = (sidx0, sidx1)
        didx = (didx0, didx1)
        rows = (rows0, rows1)
        sg = (sg0, sg1)
        ss = (ss0, ss1)
        cid = lax.axis_index("c")
        sid = lax.axis_index("s")
        row0 = sid * NPT
        nb, base = _edge_layout(cid, sid)

        for c in range(C):
            tab = t_refs[c]
            pltpu.sync_copy(zeros_r, rows0)
            _zero_slice(rows0, zsh, row0)
            plsc.subcore_barrier()

            def idx_copy(b, j):
                off = pl.multiple_of(base + b * EB, EB)
                pltpu.sync_copy(src_r.at[pl.ds(off, EB)], sidx[j])
                pltpu.sync_copy(dst_r.at[pl.ds(off, EB)], didx[j])

            idx_copy(0, 0)
            pltpu.async_copy(tab.at[sidx0], rows0, sg0)
            idx_copy(1, 1)
            pltpu.async_copy(tab.at[sidx1], rows1, sg1)

            def ring(g, carry, tab=tab, idx_copy=idx_copy):
                for j in range(2):
                    b = 2 * g + j
                    pltpu.make_async_copy(tab.at[sidx[j]], rows[j], sg[j]).wait()
                    pltpu.async_copy(rows[j], zsh.at[didx[j]], ss[j], add=True)
                    pltpu.make_async_copy(rows[j], zsh.at[didx[j]], ss[j]).wait()

                    @pl.when(b + 2 < nb)
                    def _():
                        idx_copy(b + 2, j)
                        pltpu.async_copy(tab.at[sidx[j]], rows[j], sg[j])
                return carry

            lax.fori_loop(0, nb // 2, ring, 0)

            @pl.when(nb % 2 == 1)
            def _(tab=tab):
                pltpu.make_async_copy(tab.at[sidx0], rows0, sg0).wait()
                pltpu.async_copy(rows0, zsh.at[didx0], ss0, add=True)
                pltpu.make_async_copy(rows0, zsh.at[didx0], ss0).wait()

            plsc.subcore_barrier()
            pltpu.sync_copy(zsh.at[pl.ds(row0, NPT), :],
                            out_r.at[cid, c, pl.ds(row0, NPT), :])

    return run(*tables, src, dst, zeros)


def _sc_pool(h_chunks, batch_vec):
    C = len(h_chunks)
    NBP = N // EB
    PER = -(-NBP // (NCORE * NSUB))
    GPT = G // NSUB
    zeros = jnp.zeros((EB, L), jnp.float32)

    @functools.partial(
        pl.kernel,
        out_type=jax.ShapeDtypeStruct((NCORE, C, G, L), jnp.float32),
        mesh=plsc.VectorSubcoreMesh(**_MESH),
        compiler_params=pltpu.CompilerParams(use_tc_tiling_on_sc=False),
        scratch_types=[
            pltpu.VMEM((EB,), jnp.int32),
            pltpu.VMEM((EB, L), jnp.float32),
            [pltpu.VMEM_SHARED((G, L), jnp.float32) for _ in range(C)],
        ],
    )
    def run(*refs):
        h_refs = refs[:C]
        bv_r, zeros_r, out_r = refs[C], refs[C + 1], refs[C + 2]
        didx, rows = refs[C + 3], refs[C + 4]
        zshs = refs[C + 5]
        cid = lax.axis_index("c")
        sid = lax.axis_index("s")
        wid = cid * NSUB + sid
        grow0 = sid * GPT
        pltpu.sync_copy(zeros_r, rows)
        for c in range(C):
            pltpu.sync_copy(rows.at[pl.ds(0, GPT), :],
                            zshs[c].at[pl.ds(grow0, GPT), :])
        plsc.subcore_barrier()
        for t in range(PER):
            b = wid + t * NCORE * NSUB

            @pl.when(b < NBP)
            def _():
                base = pl.multiple_of(b * EB, EB)
                pltpu.sync_copy(bv_r.at[pl.ds(base, EB)], didx)
                for c in range(C):
                    pltpu.sync_copy(h_refs[c].at[pl.ds(base, EB), :], rows)
                    pltpu.sync_copy(rows, zshs[c].at[didx], add=True)

        plsc.subcore_barrier()
        for c in range(C):
            pltpu.sync_copy(zshs[c].at[pl.ds(grow0, GPT), :],
                            out_r.at[cid, c, pl.ds(grow0, GPT), :])

    return run(*h_chunks, batch_vec, zeros)


def _tc_prep(x_p3, W1p, deg_ilv):
    def body(x_ref, w_ref, deg_ref, y_ref, dis_ref):
        d = deg_ref[0] + deg_ref[1] + 1.0
        dis = lax.rsqrt(d)
        dis_ref[...] = dis
        w = w_ref[...]
        parts = [jnp.dot(x_ref[:, j, :], w, preferred_element_type=jnp.float32)
                 for j in range(8)]
        y_ref[...] = dis * jnp.concatenate(parts, axis=1)

    return pl.pallas_call(
        body,
        grid=(IL // ILB,),
        in_specs=[
            pl.BlockSpec((ILB, 8, 128), lambda i: (i, 0, 0)),
            pl.BlockSpec((128, L), lambda i: (0, 0)),
            pl.BlockSpec((NCORE, ILB, 128), lambda i: (0, i, 0)),
        ],
        out_specs=[
            pl.BlockSpec((ILB, 128), lambda i: (i, 0)),
            pl.BlockSpec((ILB, 128), lambda i: (i, 0)),
        ],
        out_shape=[
            jax.ShapeDtypeStruct((IL, 128), jnp.float32),
            jax.ShapeDtypeStruct((IL, 128), jnp.float32),
        ],
    )(x_p3, W1p, deg_ilv)


def _tc_update(z_ilv, ychunks, dis, b128, Wbd, cout, last=False):
    cin = len(ychunks)

    def body(*refs):
        z_ref = refs[0]
        y_refs = refs[1:1 + cin]
        dis_ref = refs[1 + cin]
        b_ref = refs[2 + cin]
        k = 3 + cin
        w_ref = None
        if not last:
            w_ref = refs[k]
            k += 1
        outs = refs[k:]
        dis = dis_ref[...]
        ts = [_selu(dis * (z_ref[0, c] + z_ref[1, c] + y_refs[c][...])
                    + b_ref[c]) for c in range(cin)]
        if last:
            for c in range(cout):
                outs[c][...] = ts[c]
        else:
            for p in range(cout):
                acc = jnp.dot(ts[0], w_ref[0, p],
                              preferred_element_type=jnp.float32)
                for c in range(1, cin):
                    acc = acc + jnp.dot(ts[c], w_ref[c, p],
                                        preferred_element_type=jnp.float32)
                outs[p][...] = dis * acc

    in_specs = [pl.BlockSpec((NCORE, cin, ILB, 128), lambda i: (0, 0, i, 0))]
    in_specs += [pl.BlockSpec((ILB, 128), lambda i: (i, 0))] * cin
    in_specs += [pl.BlockSpec((ILB, 128), lambda i: (i, 0)),
                 pl.BlockSpec((cin, 128), lambda i: (0, 0))]
    args = [z_ilv, *ychunks, dis, b128]
    if not last:
        in_specs.append(pl.BlockSpec((cin, cout, 128, 128),
                                     lambda i: (0, 0, 0, 0)))
        args.append(Wbd)
    outs = pl.pallas_call(
        body,
        grid=(IL // ILB,),
        in_specs=in_specs,
        out_specs=[pl.BlockSpec((ILB, 128), lambda i: (i, 0))] * cout,
        out_shape=[jax.ShapeDtypeStruct((IL, 128), jnp.float32)] * cout,
    )(*args)
    return list(outs)


def _tc_final(pool_ilv, W1bd, b1t, W2bd, b2t):
    CIN, COUT = W1bd.shape[0], W1bd.shape[1]

    def body(p_ref, w1_ref, b1_ref, w2_ref, b2_ref, o_ref):
        ps = [p_ref[0, c] + p_ref[1, c] for c in range(CIN)]
        o1 = []
        for p in range(COUT):
            acc = jnp.dot(ps[0], w1_ref[0, p], preferred_element_type=jnp.float32)
            for c in range(1, CIN):
                acc = acc + jnp.dot(ps[c], w1_ref[c, p],
                                    preferred_element_type=jnp.float32)
            o1.append(jnp.maximum(acc + b1_ref[p], 0.0))
        acc = jnp.dot(o1[0], w2_ref[0, 0], preferred_element_type=jnp.float32)
        for p in range(1, COUT):
            acc = acc + jnp.dot(o1[p], w2_ref[p, 0],
                                preferred_element_type=jnp.float32)
        o_ref[...] = jnp.maximum(acc + b2_ref[0], 0.0)

    return pl.pallas_call(
        body,
        out_shape=jax.ShapeDtypeStruct((G // 8, 128), jnp.float32),
    )(pool_ilv, W1bd, b1t, W2bd, b2t)


def kernel(x, edge_index, batch_vec, W1, b1, W2, b2, W3, b3, W4, b4,
           Wl1, bl1, Wl2, bl2):
    src = edge_index[0]
    dst = edge_index[1]
    x_p3 = jnp.pad(x, ((0, NP - N), (0, 0))).reshape(IL, 8, 128)

    W1p = jnp.pad(W1, ((0, 0), (0, 1)))
    b1t = _tile8(jnp.pad(b1, (0, 1)))
    W2bd = _bd_expand(jnp.pad(W2, ((0, 1), (0, 12))), 1, 2)
    b2t = _tile8(jnp.pad(b2, (0, 12)))
    W3bd = _bd_expand(jnp.pad(W3, ((0, 12), (0, 5))), 2, 2)
    b3t = _tile8(jnp.pad(b3, (0, 5)))
    W4bd = _bd_expand(jnp.pad(W4, ((0, 5), (0, 12))), 2, 3)
    b4t = _tile8(jnp.pad(b4, (0, 12)))
    Wl1bd = _bd_expand(jnp.pad(Wl1, ((0, 12), (0, 0))), 3, 6)
    bl1t = _tile8(bl1)
    Wl2bd = _bd_expand(jnp.pad(Wl2, ((0, 0), (0, 4))), 6, 1)
    bl2t = _tile8(jnp.pad(bl2, (0, 4)))

    deg = _sc_deg(dst)
    y1, dis = _tc_prep(x_p3, W1p, deg.reshape(NCORE, IL, 128))
    ys = [y1]
    for Wbd, bt in [(W2bd, b1t), (W3bd, b2t), (W4bd, b3t)]:
        z = _sc_edge_pass([y.reshape(NP, L) for y in ys], src, dst)
        ys = _tc_update(z.reshape(NCORE, len(ys), IL, 128), ys, dis, bt,
                        Wbd, cout=Wbd.shape[1])
    z = _sc_edge_pass([y.reshape(NP, L) for y in ys], src, dst)
    hs = _tc_update(z.reshape(NCORE, len(ys), IL, 128), ys, dis, b4t,
                    None, cout=3, last=True)
    pool = _sc_pool([h.reshape(NP, L) for h in hs], batch_vec)
    o = _tc_final(pool.reshape(NCORE, 3, G // 8, 128), Wl1bd, bl1t,
                  Wl2bd, bl2t)
    return o.reshape(G, L)[:, :12]


# speedup vs baseline: 1.5627x; 1.1219x over previous
import functools

import jax
import jax.numpy as jnp
from jax import lax
from jax.experimental import pallas as pl
from jax.experimental.pallas import tpu as pltpu
from jax.experimental.pallas import tpu_sc as plsc

N = 100000
NP = 100352
IL = NP // 8
ILB = IL // 8
E = 1600000
G = 512
L = 16
NSUB = 16
NCORE = 2
EB = 800
NPT = N // NSUB
NB0 = 61
NB1 = 64

_MESH = dict(core_axis_name="c", subcore_axis_name="s")

_SELU_SCALE = 1.0507009873554805
_SELU_ALPHA = 1.6732632423543772


def _selu(t):
    neg = _SELU_ALPHA * (jnp.exp(jnp.minimum(t, 0.0)) - 1.0)
    return _SELU_SCALE * jnp.where(t > 0.0, t, neg)


def _bd(M):
    return jnp.kron(jnp.eye(8, dtype=jnp.float32), M)


def _bd_expand(W, cin, cout):
    return jnp.stack([
        jnp.stack([_bd(W[16 * c:16 * c + 16, 16 * p:16 * p + 16])
                   for p in range(cout)])
        for c in range(cin)])


def _tile8(b):
    c = b.shape[0] // 16
    return jnp.tile(b.reshape(c, 1, 16), (1, 8, 1)).reshape(c, 128)


def _edge_layout(cid, sid):
    nb = NB0 + (NB1 - NB0) * cid
    base = cid * (NSUB * NB0 * EB) + sid * (nb * EB)
    return nb, base


def _zero_slice(rows, zsh, row0):
    for j in range(NPT // EB):
        pltpu.sync_copy(rows, zsh.at[pl.ds(row0 + j * EB, EB), :])
    rem = NPT % EB
    pltpu.sync_copy(rows.at[pl.ds(0, rem), :],
                    zsh.at[pl.ds(row0 + (NPT // EB) * EB, rem), :])


def _sc_deg(dst):
    ones = jnp.ones((EB, L), jnp.float32)
    zeros = jnp.zeros((EB, L), jnp.float32)

    @functools.partial(
        pl.kernel,
        out_type=jax.ShapeDtypeStruct((NCORE, NP, L), jnp.float32),
        mesh=plsc.VectorSubcoreMesh(**_MESH),
        compiler_params=pltpu.CompilerParams(use_tc_tiling_on_sc=False),
        scratch_types=[
            pltpu.VMEM((EB,), jnp.int32),
            pltpu.VMEM((EB,), jnp.int32),
            pltpu.VMEM((EB, L), jnp.float32),
            pltpu.VMEM((EB, L), jnp.float32),
            pltpu.VMEM_SHARED((NP, L), jnp.float32),
            pltpu.SemaphoreType.DMA,
            pltpu.SemaphoreType.DMA,
        ],
    )
    def run(dst_r, ones_r, zeros_r, out_r, didx0, didx1, ones_v, zbuf, zsh,
            sem0, sem1):
        cid = lax.axis_index("c")
        sid = lax.axis_index("s")
        row0 = sid * NPT
        nb, base = _edge_layout(cid, sid)
        pltpu.sync_copy(ones_r, ones_v)
        pltpu.sync_copy(zeros_r, zbuf)
        _zero_slice(zbuf, zsh, row0)

        @pl.when(sid == NSUB - 1)
        def _():
            pltpu.sync_copy(zbuf.at[pl.ds(0, NP - N), :],
                            zsh.at[pl.ds(N, NP - N), :])

        plsc.subcore_barrier()

        didx = (didx0, didx1)
        sems = (sem0, sem1)

        pltpu.sync_copy(dst_r.at[pl.ds(base, EB)], didx0)
        pltpu.async_copy(ones_v, zsh.at[didx0], sem0, add=True)

        def ring(g, carry):
            for j in range(2):
                b = 2 * g + j
                q = (j + 1) % 2

                @pl.when(b + 1 < nb)
                def _():
                    off = pl.multiple_of(base + (b + 1) * EB, EB)
                    pltpu.sync_copy(dst_r.at[pl.ds(off, EB)], didx[q])
                    pltpu.async_copy(ones_v, zsh.at[didx[q]], sems[q],
                                     add=True)
                pltpu.make_async_copy(ones_v, zsh.at[didx[j]], sems[j]).wait()
            return carry

        lax.fori_loop(0, nb // 2, ring, 0)

        @pl.when(nb % 2 == 1)
        def _():
            pltpu.make_async_copy(ones_v, zsh.at[didx0], sem0).wait()

        plsc.subcore_barrier()
        pltpu.sync_copy(zsh.at[pl.ds(row0, NPT), :],
                        out_r.at[cid, pl.ds(row0, NPT), :])

        @pl.when(sid == NSUB - 1)
        def _():
            pltpu.sync_copy(zbuf.at[pl.ds(0, NP - N), :],
                            out_r.at[cid, pl.ds(N, NP - N), :])

    return run(dst, ones, zeros)


def _sc_edge_pass(tables, src, dst):
    C = len(tables)
    zeros = jnp.zeros((EB, L), jnp.float32)

    @functools.partial(
        pl.kernel,
        out_type=jax.ShapeDtypeStruct((NCORE, C, NP, L), jnp.float32),
        mesh=plsc.VectorSubcoreMesh(**_MESH),
        compiler_params=pltpu.CompilerParams(use_tc_tiling_on_sc=False),
        scratch_types=[
            pltpu.VMEM((EB,), jnp.int32),
            pltpu.VMEM((EB,), jnp.int32),
            pltpu.VMEM((EB,), jnp.int32),
            pltpu.VMEM((EB,), jnp.int32),
            pltpu.VMEM((EB, L), jnp.float32),
            pltpu.VMEM((EB, L), jnp.float32),
            pltpu.VMEM_SHARED((NP, L), jnp.float32),
            pltpu.SemaphoreType.DMA,
            pltpu.SemaphoreType.DMA,
            pltpu.SemaphoreType.DMA,
            pltpu.SemaphoreType.DMA,
        ],
    )
    def run(*refs):
        t_refs = refs[:C]
        src_r, dst_r, zeros_r, out_r = refs[C], refs[C + 1], refs[C + 2], refs[C + 3]
        (sidx0, sidx1, didx0, didx1, rows0, rows1, zsh,
         sg0, sg1, ss0, ss1) = refs[C + 4:]
        sidx = (sidx0, sidx1)
        didx = (didx0, didx1)
        rows = (rows0, rows1)
        sg = (sg0, sg1)
        ss = (ss0, ss1)
        cid = lax.axis_index("c")
        sid = lax.axis_index("s")
        row0 = sid * NPT
        nb, base = _edge_layout(cid, sid)

        for c in range(C):
            tab = t_refs[c]
            pltpu.sync_copy(zeros_r, rows0)
            _zero_slice(rows0, zsh, row0)
            plsc.subcore_barrier()

            def sidx_copy(b, j):
                off = pl.multiple_of(base + b * EB, EB)
                pltpu.sync_copy(src_r.at[pl.ds(off, EB)], sidx[j])

            def didx_copy(b, j):
                off = pl.multiple_of(base + b * EB, EB)
                pltpu.sync_copy(dst_r.at[pl.ds(off, EB)], didx[j])

            sidx_copy(0, 0)
            didx_copy(0, 0)
            pltpu.async_copy(tab.at[sidx0], rows0, sg0)
            sidx_copy(1, 1)
            didx_copy(1, 1)
            pltpu.async_copy(tab.at[sidx1], rows1, sg1)

            def ring(g, carry, tab=tab, sidx_copy=sidx_copy,
                     didx_copy=didx_copy):
                for j in range(2):
                    b = 2 * g + j
                    pltpu.make_async_copy(tab.at[sidx[j]], rows[j], sg[j]).wait()
                    pltpu.async_copy(rows[j], zsh.at[didx[j]], ss[j], add=True)

                    @pl.when(b + 2 < nb)
                    def _():
                        sidx_copy(b + 2, j)
                    pltpu.make_async_copy(rows[j], zsh.at[didx[j]], ss[j]).wait()

                    @pl.when(b + 2 < nb)
                    def _():
                        didx_copy(b + 2, j)
                        pltpu.async_copy(tab.at[sidx[j]], rows[j], sg[j])
                return carry

            lax.fori_loop(0, nb // 2, ring, 0)

            @pl.when(nb % 2 == 1)
            def _(tab=tab):
                pltpu.make_async_copy(tab.at[sidx0], rows0, sg0).wait()
                pltpu.async_copy(rows0, zsh.at[didx0], ss0, add=True)
                pltpu.make_async_copy(rows0, zsh.at[didx0], ss0).wait()

            plsc.subcore_barrier()
            pltpu.sync_copy(zsh.at[pl.ds(row0, NPT), :],
                            out_r.at[cid, c, pl.ds(row0, NPT), :])

    return run(*tables, src, dst, zeros)


def _sc_pool(h_chunks, batch_vec):
    C = len(h_chunks)
    NBP = N // EB
    PER = -(-NBP // (NCORE * NSUB))
    GPT = G // NSUB
    zeros = jnp.zeros((EB, L), jnp.float32)

    @functools.partial(
        pl.kernel,
        out_type=jax.ShapeDtypeStruct((NCORE, C, G, L), jnp.float32),
        mesh=plsc.VectorSubcoreMesh(**_MESH),
        compiler_params=pltpu.CompilerParams(use_tc_tiling_on_sc=False),
        scratch_types=[
            pltpu.VMEM((EB,), jnp.int32),
            pltpu.VMEM((EB, L), jnp.float32),
            [pltpu.VMEM_SHARED((G, L), jnp.float32) for _ in range(C)],
        ],
    )
    def run(*refs):
        h_refs = refs[:C]
        bv_r, zeros_r, out_r = refs[C], refs[C + 1], refs[C + 2]
        didx, rows = refs[C + 3], refs[C + 4]
        zshs = refs[C + 5]
        cid = lax.axis_index("c")
        sid = lax.axis_index("s")
        wid = cid * NSUB + sid
        grow0 = sid * GPT
        pltpu.sync_copy(zeros_r, rows)
        for c in range(C):
            pltpu.sync_copy(rows.at[pl.ds(0, GPT), :],
                            zshs[c].at[pl.ds(grow0, GPT), :])
        plsc.subcore_barrier()
        for t in range(PER):
            b = wid + t * NCORE * NSUB

            @pl.when(b < NBP)
            def _():
                base = pl.multiple_of(b * EB, EB)
                pltpu.sync_copy(bv_r.at[pl.ds(base, EB)], didx)
                for c in range(C):
                    pltpu.sync_copy(h_refs[c].at[pl.ds(base, EB), :], rows)
                    pltpu.sync_copy(rows, zshs[c].at[didx], add=True)

        plsc.subcore_barrier()
        for c in range(C):
            pltpu.sync_copy(zshs[c].at[pl.ds(grow0, GPT), :],
                            out_r.at[cid, c, pl.ds(grow0, GPT), :])

    return run(*h_chunks, batch_vec, zeros)


def _tc_prep(x_p3, W1p, deg_ilv):
    def body(x_ref, w_ref, deg_ref, y_ref, dis_ref):
        d = deg_ref[0] + deg_ref[1] + 1.0
        dis = lax.rsqrt(d)
        dis_ref[...] = dis
        w = w_ref[...]
        parts = [jnp.dot(x_ref[:, j, :], w, preferred_element_type=jnp.float32)
                 for j in range(8)]
        y_ref[...] = dis * jnp.concatenate(parts, axis=1)

    return pl.pallas_call(
        body,
        grid=(IL // ILB,),
        in_specs=[
            pl.BlockSpec((ILB, 8, 128), lambda i: (i, 0, 0)),
            pl.BlockSpec((128, L), lambda i: (0, 0)),
            pl.BlockSpec((NCORE, ILB, 128), lambda i: (0, i, 0)),
        ],
        out_specs=[
            pl.BlockSpec((ILB, 128), lambda i: (i, 0)),
            pl.BlockSpec((ILB, 128), lambda i: (i, 0)),
        ],
        out_shape=[
            jax.ShapeDtypeStruct((IL, 128), jnp.float32),
            jax.ShapeDtypeStruct((IL, 128), jnp.float32),
        ],
    )(x_p3, W1p, deg_ilv)


def _tc_update(z_ilv, ychunks, dis, b128, Wbd, cout, last=False):
    cin = len(ychunks)

    def body(*refs):
        z_ref = refs[0]
        y_refs = refs[1:1 + cin]
        dis_ref = refs[1 + cin]
        b_ref = refs[2 + cin]
        k = 3 + cin
        w_ref = None
        if not last:
            w_ref = refs[k]
            k += 1
        outs = refs[k:]
        dis = dis_ref[...]
        ts = [_selu(dis * (z_ref[0, c] + z_ref[1, c] + y_refs[c][...])
                    + b_ref[c]) for c in range(cin)]
        if last:
            for c in range(cout):
                outs[c][...] = ts[c]
        else:
            for p in range(cout):
                acc = jnp.dot(ts[0], w_ref[0, p],
                              preferred_element_type=jnp.float32)
                for c in range(1, cin):
                    acc = acc + jnp.dot(ts[c], w_ref[c, p],
                                        preferred_element_type=jnp.float32)
                outs[p][...] = dis * acc

    in_specs = [pl.BlockSpec((NCORE, cin, ILB, 128), lambda i: (0, 0, i, 0))]
    in_specs += [pl.BlockSpec((ILB, 128), lambda i: (i, 0))] * cin
    in_specs += [pl.BlockSpec((ILB, 128), lambda i: (i, 0)),
                 pl.BlockSpec((cin, 128), lambda i: (0, 0))]
    args = [z_ilv, *ychunks, dis, b128]
    if not last:
        in_specs.append(pl.BlockSpec((cin, cout, 128, 128),
                                     lambda i: (0, 0, 0, 0)))
        args.append(Wbd)
    outs = pl.pallas_call(
        body,
        grid=(IL // ILB,),
        in_specs=in_specs,
        out_specs=[pl.BlockSpec((ILB, 128), lambda i: (i, 0))] * cout,
        out_shape=[jax.ShapeDtypeStruct((IL, 128), jnp.float32)] * cout,
    )(*args)
    return list(outs)


def _tc_final(pool_ilv, W1bd, b1t, W2bd, b2t):
    CIN, COUT = W1bd.shape[0], W1bd.shape[1]

    def body(p_ref, w1_ref, b1_ref, w2_ref, b2_ref, o_ref):
        ps = [p_ref[0, c] + p_ref[1, c] for c in range(CIN)]
        o1 = []
        for p in range(COUT):
            acc = jnp.dot(ps[0], w1_ref[0, p], preferred_element_type=jnp.float32)
            for c in range(1, CIN):
                acc = acc + jnp.dot(ps[c], w1_ref[c, p],
                                    preferred_element_type=jnp.float32)
            o1.append(jnp.maximum(acc + b1_ref[p], 0.0))
        acc = jnp.dot(o1[0], w2_ref[0, 0], preferred_element_type=jnp.float32)
        for p in range(1, COUT):
            acc = acc + jnp.dot(o1[p], w2_ref[p, 0],
                                preferred_element_type=jnp.float32)
        o_ref[...] = jnp.maximum(acc + b2_ref[0], 0.0)

    return pl.pallas_call(
        body,
        out_shape=jax.ShapeDtypeStruct((G // 8, 128), jnp.float32),
    )(pool_ilv, W1bd, b1t, W2bd, b2t)


def kernel(x, edge_index, batch_vec, W1, b1, W2, b2, W3, b3, W4, b4,
           Wl1, bl1, Wl2, bl2):
    src = edge_index[0]
    dst = edge_index[1]
    x_p3 = jnp.pad(x, ((0, NP - N), (0, 0))).reshape(IL, 8, 128)

    W1p = jnp.pad(W1, ((0, 0), (0, 1)))
    b1t = _tile8(jnp.pad(b1, (0, 1)))
    W2bd = _bd_expand(jnp.pad(W2, ((0, 1), (0, 12))), 1, 2)
    b2t = _tile8(jnp.pad(b2, (0, 12)))
    W3bd = _bd_expand(jnp.pad(W3, ((0, 12), (0, 5))), 2, 2)
    b3t = _tile8(jnp.pad(b3, (0, 5)))
    W4bd = _bd_expand(jnp.pad(W4, ((0, 5), (0, 12))), 2, 3)
    b4t = _tile8(jnp.pad(b4, (0, 12)))
    Wl1bd = _bd_expand(jnp.pad(Wl1, ((0, 12), (0, 0))), 3, 6)
    bl1t = _tile8(bl1)
    Wl2bd = _bd_expand(jnp.pad(Wl2, ((0, 0), (0, 4))), 6, 1)
    bl2t = _tile8(jnp.pad(bl2, (0, 4)))

    deg = _sc_deg(dst)
    y1, dis = _tc_prep(x_p3, W1p, deg.reshape(NCORE, IL, 128))
    ys = [y1]
    for Wbd, bt in [(W2bd, b1t), (W3bd, b2t), (W4bd, b3t)]:
        z = _sc_edge_pass([y.reshape(NP, L) for y in ys], src, dst)
        ys = _tc_update(z.reshape(NCORE, len(ys), IL, 128), ys, dis, bt,
                        Wbd, cout=Wbd.shape[1])
    z = _sc_edge_pass([y.reshape(NP, L) for y in ys], src, dst)
    hs = _tc_update(z.reshape(NCORE, len(ys), IL, 128), ys, dis, b4t,
                    None, cout=3, last=True)
    pool = _sc_pool([h.reshape(NP, L) for h in hs], batch_vec)
    o = _tc_final(pool.reshape(NCORE, 3, G // 8, 128), Wl1bd, bl1t,
                  Wl2bd, bl2t)
    return o.reshape(G, L)[:, :12]


# speedup vs baseline: 1.6730x; 1.0706x over previous
import functools

import jax
import jax.numpy as jnp
from jax import lax
from jax.experimental import pallas as pl
from jax.experimental.pallas import tpu as pltpu
from jax.experimental.pallas import tpu_sc as plsc

N = 100000
NP = 100352
IL = NP // 8
ILB = IL // 8
E = 1600000
G = 512
L = 16
NSUB = 16
NCORE = 2
EB = 800
NPT = N // NSUB
NB0 = 61
NB1 = 64

_MESH = dict(core_axis_name="c", subcore_axis_name="s")

_SELU_SCALE = 1.0507009873554805
_SELU_ALPHA = 1.6732632423543772


def _selu(t):
    neg = _SELU_ALPHA * (jnp.exp(jnp.minimum(t, 0.0)) - 1.0)
    return _SELU_SCALE * jnp.where(t > 0.0, t, neg)


def _bd(M):
    return jnp.kron(jnp.eye(8, dtype=jnp.float32), M)


def _bd_expand(W, cin, cout):
    return jnp.stack([
        jnp.stack([_bd(W[16 * c:16 * c + 16, 16 * p:16 * p + 16])
                   for p in range(cout)])
        for c in range(cin)])


def _tile8(b):
    c = b.shape[0] // 16
    return jnp.tile(b.reshape(c, 1, 16), (1, 8, 1)).reshape(c, 128)


def _edge_layout(cid, sid):
    nb = NB0 + (NB1 - NB0) * cid
    base = cid * (NSUB * NB0 * EB) + sid * (nb * EB)
    return nb, base


def _zero_slice(rows, zsh, row0):
    for j in range(NPT // EB):
        pltpu.sync_copy(rows, zsh.at[pl.ds(row0 + j * EB, EB), :])
    rem = NPT % EB
    pltpu.sync_copy(rows.at[pl.ds(0, rem), :],
                    zsh.at[pl.ds(row0 + (NPT // EB) * EB, rem), :])


def _sc_deg(dst):
    ones = jnp.ones((EB, L), jnp.float32)
    zeros = jnp.zeros((EB, L), jnp.float32)

    @functools.partial(
        pl.kernel,
        out_type=jax.ShapeDtypeStruct((NCORE, NP, L), jnp.float32),
        mesh=plsc.VectorSubcoreMesh(**_MESH),
        compiler_params=pltpu.CompilerParams(use_tc_tiling_on_sc=False),
        scratch_types=[
            pltpu.VMEM((EB,), jnp.int32),
            pltpu.VMEM((EB,), jnp.int32),
            pltpu.VMEM((EB, L), jnp.float32),
            pltpu.VMEM((EB, L), jnp.float32),
            pltpu.VMEM_SHARED((NP, L), jnp.float32),
            pltpu.SemaphoreType.DMA,
            pltpu.SemaphoreType.DMA,
        ],
    )
    def run(dst_r, ones_r, zeros_r, out_r, didx0, didx1, ones_v, zbuf, zsh,
            sem0, sem1):
        cid = lax.axis_index("c")
        sid = lax.axis_index("s")
        row0 = sid * NPT
        nb, base = _edge_layout(cid, sid)
        pltpu.sync_copy(ones_r, ones_v)
        pltpu.sync_copy(zeros_r, zbuf)
        _zero_slice(zbuf, zsh, row0)

        @pl.when(sid == NSUB - 1)
        def _():
            pltpu.sync_copy(zbuf.at[pl.ds(0, NP - N), :],
                            zsh.at[pl.ds(N, NP - N), :])

        plsc.subcore_barrier()

        didx = (didx0, didx1)
        sems = (sem0, sem1)

        pltpu.sync_copy(dst_r.at[pl.ds(base, EB)], didx0)
        pltpu.async_copy(ones_v, zsh.at[didx0], sem0, add=True)

        def ring(g, carry):
            for j in range(2):
                b = 2 * g + j
                q = (j + 1) % 2

                @pl.when(b + 1 < nb)
                def _():
                    off = pl.multiple_of(base + (b + 1) * EB, EB)
                    pltpu.sync_copy(dst_r.at[pl.ds(off, EB)], didx[q])
                    pltpu.async_copy(ones_v, zsh.at[didx[q]], sems[q],
                                     add=True)
                pltpu.make_async_copy(ones_v, zsh.at[didx[j]], sems[j]).wait()
            return carry

        lax.fori_loop(0, nb // 2, ring, 0)

        @pl.when(nb % 2 == 1)
        def _():
            pltpu.make_async_copy(ones_v, zsh.at[didx0], sem0).wait()

        plsc.subcore_barrier()
        pltpu.sync_copy(zsh.at[pl.ds(row0, NPT), :],
                        out_r.at[cid, pl.ds(row0, NPT), :])

        @pl.when(sid == NSUB - 1)
        def _():
            pltpu.sync_copy(zbuf.at[pl.ds(0, NP - N), :],
                            out_r.at[cid, pl.ds(N, NP - N), :])

    return run(dst, ones, zeros)


def _sc_edge_pass(tables, src, dst):
    C = len(tables)
    zeros = jnp.zeros((EB, L), jnp.float32)

    @functools.partial(
        pl.kernel,
        out_type=jax.ShapeDtypeStruct((NCORE, C, NP, L), jnp.float32),
        mesh=plsc.VectorSubcoreMesh(**_MESH),
        compiler_params=pltpu.CompilerParams(use_tc_tiling_on_sc=False),
        scratch_types=[
            pltpu.VMEM((EB,), jnp.int32),
            pltpu.VMEM((EB,), jnp.int32),
            pltpu.VMEM((EB,), jnp.int32),
            pltpu.VMEM((EB,), jnp.int32),
            pltpu.VMEM((EB, L), jnp.float32),
            pltpu.VMEM((EB, L), jnp.float32),
            pltpu.VMEM_SHARED((NP, L), jnp.float32),
            pltpu.SemaphoreType.DMA,
            pltpu.SemaphoreType.DMA,
            pltpu.SemaphoreType.DMA,
            pltpu.SemaphoreType.DMA,
        ],
    )
    def run(*refs):
        t_refs = refs[:C]
        src_r, dst_r, zeros_r, out_r = refs[C], refs[C + 1], refs[C + 2], refs[C + 3]
        (sidx0, sidx1, didx0, didx1, rows0, rows1, zsh,
         sg0, sg1, ss0, ss1) = refs[C + 4:]
        sidx = (sidx0, sidx1)
        didx = (didx0, didx1)
        rows = (rows0, rows1)
        sg = (sg0, sg1)
        ss = (ss0, ss1)
        cid = lax.axis_index("c")
        sid = lax.axis_index("s")
        row0 = sid * NPT
        nb, base = _edge_layout(cid, sid)

        for c in range(C):
            tab = t_refs[c]
            pltpu.sync_copy(zeros_r, rows0)
            _zero_slice(rows0, zsh, row0)
            plsc.subcore_barrier()

            def sidx_copy(b, j):
                off = pl.multiple_of(base + b * EB, EB)
                pltpu.sync_copy(src_r.at[pl.ds(off, EB)], sidx[j])

            def didx_copy(b, j):
                off = pl.multiple_of(base + b * EB, EB)
                pltpu.sync_copy(dst_r.at[pl.ds(off, EB)], didx[j])

            sidx_copy(0, 0)
            didx_copy(0, 0)
            pltpu.async_copy(tab.at[sidx0], rows0, sg0)
            sidx_copy(1, 1)
            didx_copy(1, 1)
            pltpu.async_copy(tab.at[sidx1], rows1, sg1)

            def ring(g, carry, tab=tab, sidx_copy=sidx_copy,
                     didx_copy=didx_copy):
                for j in range(2):
                    b = 2 * g + j
                    pltpu.make_async_copy(tab.at[sidx[j]], rows[j], sg[j]).wait()
                    pltpu.async_copy(rows[j], zsh.at[didx[j]], ss[j], add=True)

                    @pl.when(b + 2 < nb)
                    def _():
                        sidx_copy(b + 2, j)
                    pltpu.make_async_copy(rows[j], zsh.at[didx[j]], ss[j]).wait()

                    @pl.when(b + 2 < nb)
                    def _():
                        pltpu.async_copy(tab.at[sidx[j]], rows[j], sg[j])
                        didx_copy(b + 2, j)
                return carry

            lax.fori_loop(0, nb // 2, ring, 0)

            @pl.when(nb % 2 == 1)
            def _(tab=tab):
                pltpu.make_async_copy(tab.at[sidx0], rows0, sg0).wait()
                pltpu.async_copy(rows0, zsh.at[didx0], ss0, add=True)
                pltpu.make_async_copy(rows0, zsh.at[didx0], ss0).wait()

            plsc.subcore_barrier()
            pltpu.sync_copy(zsh.at[pl.ds(row0, NPT), :],
                            out_r.at[cid, c, pl.ds(row0, NPT), :])

    return run(*tables, src, dst, zeros)


def _sc_pool(h_chunks, batch_vec):
    C = len(h_chunks)
    NBP = N // EB
    PER = -(-NBP // (NCORE * NSUB))
    GPT = G // NSUB
    zeros = jnp.zeros((EB, L), jnp.float32)

    @functools.partial(
        pl.kernel,
        out_type=jax.ShapeDtypeStruct((NCORE, C, G, L), jnp.float32),
        mesh=plsc.VectorSubcoreMesh(**_MESH),
        compiler_params=pltpu.CompilerParams(use_tc_tiling_on_sc=False),
        scratch_types=[
            pltpu.VMEM((EB,), jnp.int32),
            pltpu.VMEM((EB, L), jnp.float32),
            [pltpu.VMEM_SHARED((G, L), jnp.float32) for _ in range(C)],
        ],
    )
    def run(*refs):
        h_refs = refs[:C]
        bv_r, zeros_r, out_r = refs[C], refs[C + 1], refs[C + 2]
        didx, rows = refs[C + 3], refs[C + 4]
        zshs = refs[C + 5]
        cid = lax.axis_index("c")
        sid = lax.axis_index("s")
        wid = cid * NSUB + sid
        grow0 = sid * GPT
        pltpu.sync_copy(zeros_r, rows)
        for c in range(C):
            pltpu.sync_copy(rows.at[pl.ds(0, GPT), :],
                            zshs[c].at[pl.ds(grow0, GPT), :])
        plsc.subcore_barrier()
        for t in range(PER):
            b = wid + t * NCORE * NSUB

            @pl.when(b < NBP)
            def _():
                base = pl.multiple_of(b * EB, EB)
                pltpu.sync_copy(bv_r.at[pl.ds(base, EB)], didx)
                for c in range(C):
                    pltpu.sync_copy(h_refs[c].at[pl.ds(base, EB), :], rows)
                    pltpu.sync_copy(rows, zshs[c].at[didx], add=True)

        plsc.subcore_barrier()
        for c in range(C):
            pltpu.sync_copy(zshs[c].at[pl.ds(grow0, GPT), :],
                            out_r.at[cid, c, pl.ds(grow0, GPT), :])

    return run(*h_chunks, batch_vec, zeros)


def _tc_prep(x_p3, W1p, deg_ilv):
    def body(x_ref, w_ref, deg_ref, y_ref, dis_ref):
        d = deg_ref[0] + deg_ref[1] + 1.0
        dis = lax.rsqrt(d)
        dis_ref[...] = dis
        w = w_ref[...]
        parts = [jnp.dot(x_ref[:, j, :], w, preferred_element_type=jnp.float32)
                 for j in range(8)]
        y_ref[...] = dis * jnp.concatenate(parts, axis=1)

    return pl.pallas_call(
        body,
        grid=(IL // ILB,),
        in_specs=[
            pl.BlockSpec((ILB, 8, 128), lambda i: (i, 0, 0)),
            pl.BlockSpec((128, L), lambda i: (0, 0)),
            pl.BlockSpec((NCORE, ILB, 128), lambda i: (0, i, 0)),
        ],
        out_specs=[
            pl.BlockSpec((ILB, 128), lambda i: (i, 0)),
            pl.BlockSpec((ILB, 128), lambda i: (i, 0)),
        ],
        out_shape=[
            jax.ShapeDtypeStruct((IL, 128), jnp.float32),
            jax.ShapeDtypeStruct((IL, 128), jnp.float32),
        ],
    )(x_p3, W1p, deg_ilv)


def _tc_update(z_ilv, ychunks, dis, b128, Wbd, cout, last=False):
    cin = len(ychunks)

    def body(*refs):
        z_ref = refs[0]
        y_refs = refs[1:1 + cin]
        dis_ref = refs[1 + cin]
        b_ref = refs[2 + cin]
        k = 3 + cin
        w_ref = None
        if not last:
            w_ref = refs[k]
            k += 1
        outs = refs[k:]
        dis = dis_ref[...]
        ts = [_selu(dis * (z_ref[0, c] + z_ref[1, c] + y_refs[c][...])
                    + b_ref[c]) for c in range(cin)]
        if last:
            for c in range(cout):
                outs[c][...] = ts[c]
        else:
            for p in range(cout):
                acc = jnp.dot(ts[0], w_ref[0, p],
                              preferred_element_type=jnp.float32)
                for c in range(1, cin):
                    acc = acc + jnp.dot(ts[c], w_ref[c, p],
                                        preferred_element_type=jnp.float32)
                outs[p][...] = dis * acc

    in_specs = [pl.BlockSpec((NCORE, cin, ILB, 128), lambda i: (0, 0, i, 0))]
    in_specs += [pl.BlockSpec((ILB, 128), lambda i: (i, 0))] * cin
    in_specs += [pl.BlockSpec((ILB, 128), lambda i: (i, 0)),
                 pl.BlockSpec((cin, 128), lambda i: (0, 0))]
    args = [z_ilv, *ychunks, dis, b128]
    if not last:
        in_specs.append(pl.BlockSpec((cin, cout, 128, 128),
                                     lambda i: (0, 0, 0, 0)))
        args.append(Wbd)
    outs = pl.pallas_call(
        body,
        grid=(IL // ILB,),
        in_specs=in_specs,
        out_specs=[pl.BlockSpec((ILB, 128), lambda i: (i, 0))] * cout,
        out_shape=[jax.ShapeDtypeStruct((IL, 128), jnp.float32)] * cout,
    )(*args)
    return list(outs)


def _tc_final(pool_ilv, W1bd, b1t, W2bd, b2t):
    CIN, COUT = W1bd.shape[0], W1bd.shape[1]

    def body(p_ref, w1_ref, b1_ref, w2_ref, b2_ref, o_ref):
        ps = [p_ref[0, c] + p_ref[1, c] for c in range(CIN)]
        o1 = []
        for p in range(COUT):
            acc = jnp.dot(ps[0], w1_ref[0, p], preferred_element_type=jnp.float32)
            for c in range(1, CIN):
                acc = acc + jnp.dot(ps[c], w1_ref[c, p],
                                    preferred_element_type=jnp.float32)
            o1.append(jnp.maximum(acc + b1_ref[p], 0.0))
        acc = jnp.dot(o1[0], w2_ref[0, 0], preferred_element_type=jnp.float32)
        for p in range(1, COUT):
            acc = acc + jnp.dot(o1[p], w2_ref[p, 0],
                                preferred_element_type=jnp.float32)
        o_ref[...] = jnp.maximum(acc + b2_ref[0], 0.0)

    return pl.pallas_call(
        body,
        out_shape=jax.ShapeDtypeStruct((G // 8, 128), jnp.float32),
    )(pool_ilv, W1bd, b1t, W2bd, b2t)


def kernel(x, edge_index, batch_vec, W1, b1, W2, b2, W3, b3, W4, b4,
           Wl1, bl1, Wl2, bl2):
    src = edge_index[0]
    dst = edge_index[1]
    x_p3 = jnp.pad(x, ((0, NP - N), (0, 0))).reshape(IL, 8, 128)

    W1p = jnp.pad(W1, ((0, 0), (0, 1)))
    b1t = _tile8(jnp.pad(b1, (0, 1)))
    W2bd = _bd_expand(jnp.pad(W2, ((0, 1), (0, 12))), 1, 2)
    b2t = _tile8(jnp.pad(b2, (0, 12)))
    W3bd = _bd_expand(jnp.pad(W3, ((0, 12), (0, 5))), 2, 2)
    b3t = _tile8(jnp.pad(b3, (0, 5)))
    W4bd = _bd_expand(jnp.pad(W4, ((0, 5), (0, 12))), 2, 3)
    b4t = _tile8(jnp.pad(b4, (0, 12)))
    Wl1bd = _bd_expand(jnp.pad(Wl1, ((0, 12), (0, 0))), 3, 6)
    bl1t = _tile8(bl1)
    Wl2bd = _bd_expand(jnp.pad(Wl2, ((0, 0), (0, 4))), 6, 1)
    bl2t = _tile8(jnp.pad(bl2, (0, 4)))

    deg = _sc_deg(dst)
    y1, dis = _tc_prep(x_p3, W1p, deg.reshape(NCORE, IL, 128))
    ys = [y1]
    for Wbd, bt in [(W2bd, b1t), (W3bd, b2t), (W4bd, b3t)]:
        z = _sc_edge_pass([y.reshape(NP, L) for y in ys], src, dst)
        ys = _tc_update(z.reshape(NCORE, len(ys), IL, 128), ys, dis, bt,
                        Wbd, cout=Wbd.shape[1])
    z = _sc_edge_pass([y.reshape(NP, L) for y in ys], src, dst)
    hs = _tc_update(z.reshape(NCORE, len(ys), IL, 128), ys, dis, b4t,
                    None, cout=3, last=True)
    pool = _sc_pool([h.reshape(NP, L) for h in hs], batch_vec)
    o = _tc_final(pool.reshape(NCORE, 3, G // 8, 128), Wl1bd, bl1t,
                  Wl2bd, bl2t)
    return o.reshape(G, L)[:, :12]
